# baseline (speedup 1.0000x reference)
import jax
import jax.numpy as jnp
from jax import lax
from jax.experimental import pallas as pl
from jax.experimental.pallas import tpu as pltpu
from jax.experimental.pallas import tpu_sc as plsc

NUM_EMB = 1000000
NUM_TOK = 16384 * 50
DIM = 64
NC, NS = 2, 16
NW = NC * NS
SUB = 128
NSUB = 4
CHUNK = SUB * NSUB
N_CHUNKS = NUM_TOK // (NW * CHUNK)
N_SUBS = N_CHUNKS * NSUB

NB = 4096
N_TBLK = (NUM_EMB + NB - 1) // NB

B = 16384
S = 50
TB = 512


def _detile_body(tT_ref, out_ref, scratch):
    scratch[...] = jnp.transpose(tT_ref[...])
    out_ref[:, :DIM] = scratch[::2, :]
    out_ref[:, DIM:] = scratch[1::2, :]


def _detile(embeddings):
    return pl.pallas_call(
        _detile_body,
        grid=(N_TBLK,),
        in_specs=[pl.BlockSpec((DIM, NB), lambda g: (0, g))],
        out_specs=pl.BlockSpec((NB // 2, 2 * DIM), lambda g: (g, 0)),
        out_shape=jax.ShapeDtypeStruct((NUM_EMB // 2, 2 * DIM), jnp.float32),
        scratch_shapes=[pltpu.VMEM((NB, DIM), jnp.float32)],
    )(embeddings.T)


def _unpack_body(in_ref, out_ref):
    x = in_ref[...]
    for t in range(TB // 64):
        xa = x[64 * t:64 * t + 64, :DIM]
        xb = x[64 * t:64 * t + 64, DIM:]
        y = jnp.concatenate([xa.T, xb.T], axis=1)
        out_ref[0, :, t] = y.reshape(8, 8, 128)


def _unpack(rows128):
    n_tau = B // (2 * TB)
    return pl.pallas_call(
        _unpack_body,
        grid=(S, n_tau),
        in_specs=[pl.BlockSpec((TB, 128), lambda s, tau: (s * n_tau + tau, 0))],
        out_specs=pl.BlockSpec((1, 8, TB // 64, 8, 128),
                               lambda s, tau: (s, 0, tau, 0, 0)),
        out_shape=jax.ShapeDtypeStruct((S, 8, B // 128, 8, 128), jnp.float32),
    )(rows128)


def _emb_body(idx_hbm, table_hbm, out_hbm,
              idx_v, rows0, rows1, sg0, sg1, so0, so1):
    wid = lax.axis_index("s") * NC + lax.axis_index("c")
    out_base = wid * N_CHUNKS * CHUNK

    pltpu.sync_copy(idx_hbm.at[wid], idx_v)

    def issue_gather(i, rows, sem):
        return [
            pltpu.async_copy(
                table_hbm.at[idx_v.at[i * NSUB + j]],
                rows.at[pl.ds(j * SUB, SUB)],
                sem,
            )
            for j in range(NSUB)
        ]

    def wait_all(cps):
        for cp in cps:
            cp.wait()

    def issue_store(i, rows, sem):
        return pltpu.async_copy(
            rows, out_hbm.at[pl.ds(out_base + i * CHUNK, CHUNK)], sem)

    wait_all(issue_gather(0, rows0, sg0))

    def step(t, carry):
        i0 = t * 2
        g1 = issue_gather(i0 + 1, rows1, sg1)
        issue_store(i0, rows0, so0).wait()

        @pl.when(i0 + 2 < N_CHUNKS)
        def _():
            g2 = issue_gather(i0 + 2, rows0, sg0)
            wait_all(g1)
            issue_store(i0 + 1, rows1, so1).wait()
            wait_all(g2)

        @pl.when(i0 + 2 >= N_CHUNKS)
        def _():
            wait_all(g1)
            issue_store(i0 + 1, rows1, so1).wait()

        return carry

    lax.fori_loop(0, N_CHUNKS // 2, step, 0)


@jax.jit
def _embedding_lookup(idx, table):
    grouped = idx.reshape(NW, N_SUBS, SUB)
    t64 = _detile(table).reshape(NUM_EMB, DIM)
    return pl.kernel(
        _emb_body,
        out_type=jax.ShapeDtypeStruct((NUM_TOK, DIM), jnp.float32),
        mesh=plsc.VectorSubcoreMesh(core_axis_name="c", subcore_axis_name="s"),
        compiler_params=pltpu.CompilerParams(use_tc_tiling_on_sc=False),
        scratch_types=[
            pltpu.VMEM((N_SUBS, SUB), jnp.int32),
            pltpu.VMEM((CHUNK, DIM), jnp.float32),
            pltpu.VMEM((CHUNK, DIM), jnp.float32),
            pltpu.SemaphoreType.DMA,
            pltpu.SemaphoreType.DMA,
            pltpu.SemaphoreType.DMA,
            pltpu.SemaphoreType.DMA,
        ],
    )(grouped, t64)


def kernel(token_ids, embeddings):
    perm = (token_ids.T.astype(jnp.int32)
            .reshape(S, B // 128, 2, 64)
            .transpose(0, 1, 3, 2)
            .reshape(-1))
    rows = _embedding_lookup(perm, embeddings)
    out5 = _unpack(rows.reshape(NUM_TOK // 2, 128))
    o = out5.transpose(0, 1, 3, 2, 4).reshape(S, DIM, B)
    return o.transpose(2, 0, 1)


# speedup vs baseline: 1.1901x; 1.1901x over previous
import jax
import jax.numpy as jnp
from jax import lax
from jax.experimental import pallas as pl
from jax.experimental.pallas import tpu as pltpu
from jax.experimental.pallas import tpu_sc as plsc

NUM_EMB = 1000000
NUM_TOK = 16384 * 50
DIM = 64
NC, NS = 2, 16
NW = NC * NS
SUB = 128
NSUB = 4
CHUNK = SUB * NSUB
N_CHUNKS = NUM_TOK // (NW * CHUNK)
N_SUBS = N_CHUNKS * NSUB

NB = 4096
N_TBLK = (NUM_EMB + NB - 1) // NB

B = 16384
S = 50
TB = 512


def _detile_body(tT_ref, out_ref, scratch):
    scratch[...] = jnp.transpose(tT_ref[...])
    out_ref[:, :DIM] = scratch[::2, :]
    out_ref[:, DIM:] = scratch[1::2, :]


def _detile(embeddings):
    return pl.pallas_call(
        _detile_body,
        grid=(N_TBLK,),
        in_specs=[pl.BlockSpec((DIM, NB), lambda g: (0, g))],
        out_specs=pl.BlockSpec((NB // 2, 2 * DIM), lambda g: (g, 0)),
        out_shape=jax.ShapeDtypeStruct((NUM_EMB // 2, 2 * DIM), jnp.float32),
        scratch_shapes=[pltpu.VMEM((NB, DIM), jnp.float32)],
    )(embeddings.T)


def _unpack_body(in_ref, out_ref):
    x = in_ref[...]
    y = jnp.transpose(x.reshape(TB // 64, 64, 128), (0, 2, 1))
    for t in range(TB // 64):
        out_ref[0, :, t] = jnp.concatenate(
            [y[t, :DIM, :], y[t, DIM:, :]], axis=1).reshape(8, 8, 128)


def _unpack(rows128):
    n_tau = B // (2 * TB)
    return pl.pallas_call(
        _unpack_body,
        grid=(S, n_tau),
        in_specs=[pl.BlockSpec((TB, 128), lambda s, tau: (s * n_tau + tau, 0))],
        out_specs=pl.BlockSpec((1, 8, TB // 64, 8, 128),
                               lambda s, tau: (s, 0, tau, 0, 0)),
        out_shape=jax.ShapeDtypeStruct((S, 8, B // 128, 8, 128), jnp.float32),
    )(rows128)


def _emb_body(idx_hbm, table_hbm, out_hbm,
              idx_v, rows0, rows1, sg0, sg1, so0, so1):
    wid = lax.axis_index("s") * NC + lax.axis_index("c")
    out_base = wid * N_CHUNKS * CHUNK

    pltpu.sync_copy(idx_hbm.at[wid], idx_v)

    def issue_gather(i, rows, sem):
        return [
            pltpu.async_copy(
                table_hbm.at[idx_v.at[i * NSUB + j]],
                rows.at[pl.ds(j * SUB, SUB)],
                sem,
            )
            for j in range(NSUB)
        ]

    def wait_all(cps):
        for cp in cps:
            cp.wait()

    def issue_store(i, rows, sem):
        return pltpu.async_copy(
            rows, out_hbm.at[pl.ds(out_base + i * CHUNK, CHUNK)], sem)

    wait_all(issue_gather(0, rows0, sg0))

    def step(t, carry):
        i0 = t * 2
        g1 = issue_gather(i0 + 1, rows1, sg1)
        issue_store(i0, rows0, so0).wait()

        @pl.when(i0 + 2 < N_CHUNKS)
        def _():
            g2 = issue_gather(i0 + 2, rows0, sg0)
            wait_all(g1)
            issue_store(i0 + 1, rows1, so1).wait()
            wait_all(g2)

        @pl.when(i0 + 2 >= N_CHUNKS)
        def _():
            wait_all(g1)
            issue_store(i0 + 1, rows1, so1).wait()

        return carry

    lax.fori_loop(0, N_CHUNKS // 2, step, 0)


@jax.jit
def _embedding_lookup(idx, table):
    grouped = idx.reshape(NW, N_SUBS, SUB)
    t64 = _detile(table).reshape(NUM_EMB, DIM)
    return pl.kernel(
        _emb_body,
        out_type=jax.ShapeDtypeStruct((NUM_TOK, DIM), jnp.float32),
        mesh=plsc.VectorSubcoreMesh(core_axis_name="c", subcore_axis_name="s"),
        compiler_params=pltpu.CompilerParams(use_tc_tiling_on_sc=False),
        scratch_types=[
            pltpu.VMEM((N_SUBS, SUB), jnp.int32),
            pltpu.VMEM((CHUNK, DIM), jnp.float32),
            pltpu.VMEM((CHUNK, DIM), jnp.float32),
            pltpu.SemaphoreType.DMA,
            pltpu.SemaphoreType.DMA,
            pltpu.SemaphoreType.DMA,
            pltpu.SemaphoreType.DMA,
        ],
    )(grouped, t64)


def kernel(token_ids, embeddings):
    perm = (token_ids.T.astype(jnp.int32)
            .reshape(S, B // 128, 2, 64)
            .transpose(0, 1, 3, 2)
            .reshape(-1))
    rows = _embedding_lookup(perm, embeddings)
    out5 = _unpack(rows.reshape(NUM_TOK // 2, 128))
    o = out5.transpose(0, 1, 3, 2, 4).reshape(S, DIM, B)
    return o.transpose(2, 0, 1)


# speedup vs baseline: 1.4506x; 1.2189x over previous
import jax
import jax.numpy as jnp
from jax import lax
from jax.experimental import pallas as pl
from jax.experimental.pallas import tpu as pltpu
from jax.experimental.pallas import tpu_sc as plsc

NUM_EMB = 1000000
NUM_TOK = 16384 * 50
DIM = 64
NC, NS = 2, 16
NW = NC * NS
SUB = 128
NSUB = 4
CHUNK = SUB * NSUB
N_CHUNKS = NUM_TOK // (NW * CHUNK)
N_SUBS = N_CHUNKS * NSUB

NB = 4096
N_TBLK = (NUM_EMB + NB - 1) // NB

B = 16384
S = 50
TB = 1024


def _detile_body(tT_ref, out_ref, scratch):
    scratch[...] = jnp.transpose(tT_ref[...])
    out_ref[:, :DIM] = scratch[::2, :]
    out_ref[:, DIM:] = scratch[1::2, :]


def _detile(embeddings):
    return pl.pallas_call(
        _detile_body,
        grid=(N_TBLK,),
        in_specs=[pl.BlockSpec((DIM, NB), lambda g: (0, g))],
        out_specs=pl.BlockSpec((NB // 2, 2 * DIM), lambda g: (g, 0)),
        out_shape=jax.ShapeDtypeStruct((NUM_EMB // 2, 2 * DIM), jnp.float32),
        scratch_shapes=[pltpu.VMEM((NB, DIM), jnp.float32)],
    )(embeddings.T)


def _unpack_body(in_ref, out_ref):
    x = in_ref[...]
    y = jnp.transpose(x.reshape(TB // 64, 64, 128), (0, 2, 1))
    for t in range(TB // 64):
        out_ref[0, :, t, :, :DIM] = y[t, :DIM, :].reshape(8, 8, DIM)
        out_ref[0, :, t, :, DIM:] = y[t, DIM:, :].reshape(8, 8, DIM)


def _unpack(rows128):
    n_tau = B // (2 * TB)
    return pl.pallas_call(
        _unpack_body,
        grid=(S, n_tau),
        in_specs=[pl.BlockSpec((TB, 128), lambda s, tau: (s * n_tau + tau, 0))],
        out_specs=pl.BlockSpec((1, 8, TB // 64, 8, 128),
                               lambda s, tau: (s, 0, tau, 0, 0)),
        out_shape=jax.ShapeDtypeStruct((S, 8, B // 128, 8, 128), jnp.float32),
    )(rows128)


def _emb_body(idx_hbm, table_hbm, out_hbm,
              idx_v, rows0, rows1, sg0, sg1, so0, so1):
    wid = lax.axis_index("s") * NC + lax.axis_index("c")
    out_base = wid * N_CHUNKS * CHUNK

    pltpu.sync_copy(idx_hbm.at[wid], idx_v)

    def issue_gather(i, rows, sem):
        return [
            pltpu.async_copy(
                table_hbm.at[idx_v.at[i * NSUB + j]],
                rows.at[pl.ds(j * SUB, SUB)],
                sem,
            )
            for j in range(NSUB)
        ]

    def wait_all(cps):
        for cp in cps:
            cp.wait()

    def issue_store(i, rows, sem):
        return pltpu.async_copy(
            rows, out_hbm.at[pl.ds(out_base + i * CHUNK, CHUNK)], sem)

    wait_all(issue_gather(0, rows0, sg0))

    def step(t, carry):
        i0 = t * 2
        g1 = issue_gather(i0 + 1, rows1, sg1)
        issue_store(i0, rows0, so0).wait()

        @pl.when(i0 + 2 < N_CHUNKS)
        def _():
            g2 = issue_gather(i0 + 2, rows0, sg0)
            wait_all(g1)
            issue_store(i0 + 1, rows1, so1).wait()
            wait_all(g2)

        @pl.when(i0 + 2 >= N_CHUNKS)
        def _():
            wait_all(g1)
            issue_store(i0 + 1, rows1, so1).wait()

        return carry

    lax.fori_loop(0, N_CHUNKS // 2, step, 0)


@jax.jit
def _embedding_lookup(idx, table):
    grouped = idx.reshape(NW, N_SUBS, SUB)
    t64 = _detile(table).reshape(NUM_EMB, DIM)
    return pl.kernel(
        _emb_body,
        out_type=jax.ShapeDtypeStruct((NUM_TOK, DIM), jnp.float32),
        mesh=plsc.VectorSubcoreMesh(core_axis_name="c", subcore_axis_name="s"),
        compiler_params=pltpu.CompilerParams(use_tc_tiling_on_sc=False),
        scratch_types=[
            pltpu.VMEM((N_SUBS, SUB), jnp.int32),
            pltpu.VMEM((CHUNK, DIM), jnp.float32),
            pltpu.VMEM((CHUNK, DIM), jnp.float32),
            pltpu.SemaphoreType.DMA,
            pltpu.SemaphoreType.DMA,
            pltpu.SemaphoreType.DMA,
            pltpu.SemaphoreType.DMA,
        ],
    )(grouped, t64)


def kernel(token_ids, embeddings):
    perm = (token_ids.T.astype(jnp.int32)
            .reshape(S, B // 128, 2, 64)
            .transpose(0, 1, 3, 2)
            .reshape(-1))
    rows = _embedding_lookup(perm, embeddings)
    out5 = _unpack(rows.reshape(NUM_TOK // 2, 128))
    o = out5.transpose(0, 1, 3, 2, 4).reshape(S, DIM, B)
    return o.transpose(2, 0, 1)


# speedup vs baseline: 1.7717x; 1.2214x over previous
import jax
import jax.numpy as jnp
from jax import lax
from jax.experimental import pallas as pl
from jax.experimental.pallas import tpu as pltpu
from jax.experimental.pallas import tpu_sc as plsc

NUM_EMB = 1000000
NUM_TOK = 16384 * 50
DIM = 64
NC, NS = 2, 16
NW = NC * NS
SUB = 128
NSUB = 4
CHUNK = SUB * NSUB
N_CHUNKS = NUM_TOK // (NW * CHUNK)
N_SUBS = N_CHUNKS * NSUB

NB = 8192
N_TBLK = (NUM_EMB + NB - 1) // NB

B = 16384
S = 50
TB = 2048


def _detile_body(tT_ref, out_ref, scratch):
    scratch[...] = jnp.transpose(tT_ref[...])
    out_ref[:, :DIM] = scratch[::2, :]
    out_ref[:, DIM:] = scratch[1::2, :]


def _detile(embeddings):
    return pl.pallas_call(
        _detile_body,
        grid=(N_TBLK,),
        in_specs=[pl.BlockSpec((DIM, NB), lambda g: (0, g))],
        out_specs=pl.BlockSpec((NB // 2, 2 * DIM), lambda g: (g, 0)),
        out_shape=jax.ShapeDtypeStruct((NUM_EMB // 2, 2 * DIM), jnp.float32),
        scratch_shapes=[pltpu.VMEM((NB, DIM), jnp.float32)],
    )(embeddings.T)


def _unpack_body(in_ref, out_ref):
    x = in_ref[...]
    y = jnp.transpose(x.reshape(TB // 64, 64, 128), (0, 2, 1))
    for t in range(TB // 64):
        out_ref[0, :, t, :, :DIM] = y[t, :DIM, :].reshape(8, 8, DIM)
        out_ref[0, :, t, :, DIM:] = y[t, DIM:, :].reshape(8, 8, DIM)


def _unpack(rows128):
    n_tau = B // (2 * TB)
    return pl.pallas_call(
        _unpack_body,
        grid=(S, n_tau),
        in_specs=[pl.BlockSpec((TB, 128), lambda s, tau: (s * n_tau + tau, 0))],
        out_specs=pl.BlockSpec((1, 8, TB // 64, 8, 128),
                               lambda s, tau: (s, 0, tau, 0, 0)),
        out_shape=jax.ShapeDtypeStruct((S, 8, B // 128, 8, 128), jnp.float32),
    )(rows128)


def _emb_body(idx_hbm, table_hbm, out_hbm,
              idx_v, rows0, rows1, sg0, sg1, so0, so1):
    wid = lax.axis_index("s") * NC + lax.axis_index("c")
    out_base = wid * N_CHUNKS * CHUNK

    pltpu.sync_copy(idx_hbm.at[wid], idx_v)

    def issue_gather(i, rows, sem):
        return [
            pltpu.async_copy(
                table_hbm.at[idx_v.at[i * NSUB + j]],
                rows.at[pl.ds(j * SUB, SUB)],
                sem,
            )
            for j in range(NSUB)
        ]

    def wait_all(cps):
        for cp in cps:
            cp.wait()

    def issue_store(i, rows, sem):
        return pltpu.async_copy(
            rows, out_hbm.at[pl.ds(out_base + i * CHUNK, CHUNK)], sem)

    wait_all(issue_gather(0, rows0, sg0))

    def step(t, carry):
        i0 = t * 2
        g1 = issue_gather(i0 + 1, rows1, sg1)
        issue_store(i0, rows0, so0).wait()

        @pl.when(i0 + 2 < N_CHUNKS)
        def _():
            g2 = issue_gather(i0 + 2, rows0, sg0)
            wait_all(g1)
            issue_store(i0 + 1, rows1, so1).wait()
            wait_all(g2)

        @pl.when(i0 + 2 >= N_CHUNKS)
        def _():
            wait_all(g1)
            issue_store(i0 + 1, rows1, so1).wait()

        return carry

    lax.fori_loop(0, N_CHUNKS // 2, step, 0)


@jax.jit
def _embedding_lookup(idx, table):
    grouped = idx.reshape(NW, N_SUBS, SUB)
    t64 = _detile(table).reshape(NUM_EMB, DIM)
    return pl.kernel(
        _emb_body,
        out_type=jax.ShapeDtypeStruct((NUM_TOK, DIM), jnp.float32),
        mesh=plsc.VectorSubcoreMesh(core_axis_name="c", subcore_axis_name="s"),
        compiler_params=pltpu.CompilerParams(use_tc_tiling_on_sc=False),
        scratch_types=[
            pltpu.VMEM((N_SUBS, SUB), jnp.int32),
            pltpu.VMEM((CHUNK, DIM), jnp.float32),
            pltpu.VMEM((CHUNK, DIM), jnp.float32),
            pltpu.SemaphoreType.DMA,
            pltpu.SemaphoreType.DMA,
            pltpu.SemaphoreType.DMA,
            pltpu.SemaphoreType.DMA,
        ],
    )(grouped, t64)


def kernel(token_ids, embeddings):
    perm = (token_ids.T.astype(jnp.int32)
            .reshape(S, B // 128, 2, 64)
            .transpose(0, 1, 3, 2)
            .reshape(-1))
    rows = _embedding_lookup(perm, embeddings)
    out5 = _unpack(rows.reshape(NUM_TOK // 2, 128))
    o = out5.transpose(0, 1, 3, 2, 4).reshape(S, DIM, B)
    return o.transpose(2, 0, 1)


# speedup vs baseline: 1.9603x; 1.1064x over previous
import jax
import jax.numpy as jnp
from jax import lax
from jax.experimental import pallas as pl
from jax.experimental.pallas import tpu as pltpu
from jax.experimental.pallas import tpu_sc as plsc

NUM_EMB = 1000000
NUM_TOK = 16384 * 50
DIM = 64
NC, NS = 2, 16
NW = NC * NS
SUB = 128
NSUB = 4
CHUNK = SUB * NSUB
N_CHUNKS = NUM_TOK // (NW * CHUNK)
N_SUBS = N_CHUNKS * NSUB

NB = 16384
N_TBLK = (NUM_EMB + NB - 1) // NB

B = 16384
S = 50
TB = 4096


def _detile_body(tT_ref, out_ref, scratch):
    scratch[...] = jnp.transpose(tT_ref[...])
    out_ref[:, :DIM] = scratch[::2, :]
    out_ref[:, DIM:] = scratch[1::2, :]


def _detile(embeddings):
    return pl.pallas_call(
        _detile_body,
        grid=(N_TBLK,),
        in_specs=[pl.BlockSpec((DIM, NB), lambda g: (0, g))],
        out_specs=pl.BlockSpec((NB // 2, 2 * DIM), lambda g: (g, 0)),
        out_shape=jax.ShapeDtypeStruct((NUM_EMB // 2, 2 * DIM), jnp.float32),
        scratch_shapes=[pltpu.VMEM((NB, DIM), jnp.float32)],
    )(embeddings.T)


def _unpack_body(in_ref, out_ref):
    x = in_ref[...]
    y = jnp.transpose(x.reshape(TB // 64, 64, 128), (0, 2, 1))
    for t in range(TB // 64):
        out_ref[0, :, t, :, :DIM] = y[t, :DIM, :].reshape(8, 8, DIM)
        out_ref[0, :, t, :, DIM:] = y[t, DIM:, :].reshape(8, 8, DIM)


def _unpack(rows128):
    n_tau = B // (2 * TB)
    return pl.pallas_call(
        _unpack_body,
        grid=(S, n_tau),
        in_specs=[pl.BlockSpec((TB, 128), lambda s, tau: (s * n_tau + tau, 0))],
        out_specs=pl.BlockSpec((1, 8, TB // 64, 8, 128),
                               lambda s, tau: (s, 0, tau, 0, 0)),
        out_shape=jax.ShapeDtypeStruct((S, 8, B // 128, 8, 128), jnp.float32),
    )(rows128)


def _emb_body(idx_hbm, table_hbm, out_hbm,
              idx_v, rows0, rows1, sg0, sg1, so0, so1):
    wid = lax.axis_index("s") * NC + lax.axis_index("c")
    out_base = wid * N_CHUNKS * CHUNK

    pltpu.sync_copy(idx_hbm.at[wid], idx_v)

    def issue_gather(i, rows, sem):
        return [
            pltpu.async_copy(
                table_hbm.at[idx_v.at[i * NSUB + j]],
                rows.at[pl.ds(j * SUB, SUB)],
                sem,
            )
            for j in range(NSUB)
        ]

    def wait_all(cps):
        for cp in cps:
            cp.wait()

    def issue_store(i, rows, sem):
        return pltpu.async_copy(
            rows, out_hbm.at[pl.ds(out_base + i * CHUNK, CHUNK)], sem)

    wait_all(issue_gather(0, rows0, sg0))

    def step(t, carry):
        i0 = t * 2
        g1 = issue_gather(i0 + 1, rows1, sg1)
        issue_store(i0, rows0, so0).wait()

        @pl.when(i0 + 2 < N_CHUNKS)
        def _():
            g2 = issue_gather(i0 + 2, rows0, sg0)
            wait_all(g1)
            issue_store(i0 + 1, rows1, so1).wait()
            wait_all(g2)

        @pl.when(i0 + 2 >= N_CHUNKS)
        def _():
            wait_all(g1)
            issue_store(i0 + 1, rows1, so1).wait()

        return carry

    lax.fori_loop(0, N_CHUNKS // 2, step, 0)


@jax.jit
def _embedding_lookup(idx, table):
    grouped = idx.reshape(NW, N_SUBS, SUB)
    t64 = _detile(table).reshape(NUM_EMB, DIM)
    return pl.kernel(
        _emb_body,
        out_type=jax.ShapeDtypeStruct((NUM_TOK, DIM), jnp.float32),
        mesh=plsc.VectorSubcoreMesh(core_axis_name="c", subcore_axis_name="s"),
        compiler_params=pltpu.CompilerParams(use_tc_tiling_on_sc=False),
        scratch_types=[
            pltpu.VMEM((N_SUBS, SUB), jnp.int32),
            pltpu.VMEM((CHUNK, DIM), jnp.float32),
            pltpu.VMEM((CHUNK, DIM), jnp.float32),
            pltpu.SemaphoreType.DMA,
            pltpu.SemaphoreType.DMA,
            pltpu.SemaphoreType.DMA,
            pltpu.SemaphoreType.DMA,
        ],
    )(grouped, t64)


def kernel(token_ids, embeddings):
    perm = (token_ids.T.astype(jnp.int32)
            .reshape(S, B // 128, 2, 64)
            .transpose(0, 1, 3, 2)
            .reshape(-1))
    rows = _embedding_lookup(perm, embeddings)
    out5 = _unpack(rows.reshape(NUM_TOK // 2, 128))
    o = out5.transpose(0, 1, 3, 2, 4).reshape(S, DIM, B)
    return o.transpose(2, 0, 1)
